# baseline (device time: 153409 ns/iter reference)
import jax
import jax.numpy as jnp
from jax import lax
from jax.experimental import pallas as pl
from jax.experimental.pallas import tpu as pltpu

N_DEV = 4


def kernel(partial, gamma):
    _, M, D = partial.shape
    x = partial.reshape(M, D)
    g = gamma.reshape(1, D)
    m_per = M // N_DEV

    def body(x_ref, g_ref, out_ref, comm_ref, send_sems, recv_sems):
        my = lax.axis_index("i")
        left = (my - 1) % N_DEV
        right = (my + 1) % N_DEV

        barrier_sem = pltpu.get_barrier_semaphore()
        for nbr in (left, right):
            pl.semaphore_signal(
                barrier_sem, inc=1,
                device_id=(nbr,), device_id_type=pl.DeviceIdType.MESH,
            )
        pl.semaphore_wait(barrier_sem, 2)

        c0 = (my - 1) % N_DEV
        comm_ref[0, :, :] = x_ref[pl.ds(c0 * m_per, m_per), :]

        for h in range(N_DEV - 1):
            rdma = pltpu.make_async_remote_copy(
                src_ref=comm_ref.at[h],
                dst_ref=comm_ref.at[h + 1],
                send_sem=send_sems.at[h],
                recv_sem=recv_sems.at[h],
                device_id=(right,),
                device_id_type=pl.DeviceIdType.MESH,
            )
            rdma.start()
            rdma.wait()

            c = (my - 2 - h) % N_DEV
            local = x_ref[pl.ds(c * m_per, m_per), :]
            if h < N_DEV - 2:
                comm_ref[h + 1, :, :] = comm_ref[h + 1, :, :] + local
            else:
                y = comm_ref[h + 1, :, :] + local
                inv = lax.rsqrt(jnp.mean(y * y, axis=-1, keepdims=True) + 1e-6)
                out_ref[:, :] = y * inv * g_ref[:, :]

    return pl.pallas_call(
        body,
        out_shape=jax.ShapeDtypeStruct((m_per, D), jnp.float32),
        in_specs=[
            pl.BlockSpec(memory_space=pltpu.VMEM),
            pl.BlockSpec(memory_space=pltpu.VMEM),
        ],
        out_specs=pl.BlockSpec(memory_space=pltpu.VMEM),
        scratch_shapes=[
            pltpu.VMEM((N_DEV, m_per, D), jnp.float32),
            pltpu.SemaphoreType.DMA((N_DEV - 1,)),
            pltpu.SemaphoreType.DMA((N_DEV - 1,)),
        ],
        compiler_params=pltpu.CompilerParams(collective_id=0),
    )(x, g)


# device time: 86112 ns/iter; 1.7815x vs baseline; 1.7815x over previous
import jax
import jax.numpy as jnp
from jax import lax
from jax.experimental import pallas as pl
from jax.experimental.pallas import tpu as pltpu

N_DEV = 4


def kernel(partial, gamma):
    _, M, D = partial.shape
    x = partial.reshape(M, D)
    g = gamma.reshape(1, D)
    m_per = M // N_DEV
    d_half = D // 2

    def body(
        x_ref, g_ref, out_ref,
        comm_r, comm_l, send_r, recv_r, send_l, recv_l,
    ):
        my = lax.axis_index("i")
        left = (my - 1) % N_DEV
        right = (my + 1) % N_DEV

        barrier_sem = pltpu.get_barrier_semaphore()
        for nbr in (left, right):
            pl.semaphore_signal(
                barrier_sem, inc=1,
                device_id=(nbr,), device_id_type=pl.DeviceIdType.MESH,
            )
        pl.semaphore_wait(barrier_sem, 2)

        cr = (my - 1) % N_DEV
        cl = (my + 1) % N_DEV
        comm_r[0, :, :] = x_ref[pl.ds(cr * m_per, m_per), :d_half]
        comm_l[0, :, :] = x_ref[pl.ds(cl * m_per, m_per), d_half:]

        for h in range(N_DEV - 1):
            rdma_r = pltpu.make_async_remote_copy(
                src_ref=comm_r.at[h],
                dst_ref=comm_r.at[h + 1],
                send_sem=send_r.at[h],
                recv_sem=recv_r.at[h],
                device_id=(right,),
                device_id_type=pl.DeviceIdType.MESH,
            )
            rdma_l = pltpu.make_async_remote_copy(
                src_ref=comm_l.at[h],
                dst_ref=comm_l.at[h + 1],
                send_sem=send_l.at[h],
                recv_sem=recv_l.at[h],
                device_id=(left,),
                device_id_type=pl.DeviceIdType.MESH,
            )
            rdma_r.start()
            rdma_l.start()
            rdma_r.wait()
            rdma_l.wait()

            c_r = (my - 2 - h) % N_DEV
            c_l = (my + 2 + h) % N_DEV
            loc_r = x_ref[pl.ds(c_r * m_per, m_per), :d_half]
            loc_l = x_ref[pl.ds(c_l * m_per, m_per), d_half:]
            if h < N_DEV - 2:
                comm_r[h + 1, :, :] = comm_r[h + 1, :, :] + loc_r
                comm_l[h + 1, :, :] = comm_l[h + 1, :, :] + loc_l
            else:
                y = jnp.concatenate(
                    [comm_r[h + 1, :, :] + loc_r, comm_l[h + 1, :, :] + loc_l],
                    axis=1,
                )
                inv = lax.rsqrt(jnp.mean(y * y, axis=-1, keepdims=True) + 1e-6)
                out_ref[:, :] = y * inv * g_ref[:, :]

    return pl.pallas_call(
        body,
        out_shape=jax.ShapeDtypeStruct((m_per, D), jnp.float32),
        in_specs=[
            pl.BlockSpec(memory_space=pltpu.VMEM),
            pl.BlockSpec(memory_space=pltpu.VMEM),
        ],
        out_specs=pl.BlockSpec(memory_space=pltpu.VMEM),
        scratch_shapes=[
            pltpu.VMEM((N_DEV, m_per, d_half), jnp.float32),
            pltpu.VMEM((N_DEV, m_per, d_half), jnp.float32),
            pltpu.SemaphoreType.DMA((N_DEV - 1,)),
            pltpu.SemaphoreType.DMA((N_DEV - 1,)),
            pltpu.SemaphoreType.DMA((N_DEV - 1,)),
            pltpu.SemaphoreType.DMA((N_DEV - 1,)),
        ],
        compiler_params=pltpu.CompilerParams(collective_id=0),
    )(x, g)


# device time: 81418 ns/iter; 1.8842x vs baseline; 1.0577x over previous
import jax
import jax.numpy as jnp
from jax import lax
from jax.experimental import pallas as pl
from jax.experimental.pallas import tpu as pltpu

N_DEV = 4
S = 2


def kernel(partial, gamma):
    _, M, D = partial.shape
    x = partial.reshape(M, D)
    g = gamma.reshape(1, D)
    m_per = M // N_DEV
    d_half = D // 2
    m_sub = m_per // S

    def body(
        x_ref, g_ref, out_ref,
        comm_r, comm_l, send_r, recv_r, send_l, recv_l,
    ):
        my = lax.axis_index("i")
        left = (my - 1) % N_DEV
        right = (my + 1) % N_DEV

        barrier_sem = pltpu.get_barrier_semaphore()
        for nbr in (left, right):
            pl.semaphore_signal(
                barrier_sem, inc=1,
                device_id=(nbr,), device_id_type=pl.DeviceIdType.MESH,
            )
        pl.semaphore_wait(barrier_sem, 2)

        def mk(comm, sends, recvs, h, k, tgt):
            rows = pl.ds(k * m_sub, m_sub)
            return pltpu.make_async_remote_copy(
                src_ref=comm.at[h, rows],
                dst_ref=comm.at[h + 1, rows],
                send_sem=sends.at[h, k],
                recv_sem=recvs.at[h, k],
                device_id=(tgt,),
                device_id_type=pl.DeviceIdType.MESH,
            )

        cr = (my - 1) % N_DEV
        cl = (my + 1) % N_DEV
        descs = {}
        for k in range(S):
            sub = slice(k * m_sub, (k + 1) * m_sub)
            comm_r[0, sub, :] = x_ref[pl.ds(cr * m_per + k * m_sub, m_sub), :d_half]
            d = mk(comm_r, send_r, recv_r, 0, k, right)
            d.start()
            descs[("r", 0, k)] = d
            comm_l[0, sub, :] = x_ref[pl.ds(cl * m_per + k * m_sub, m_sub), d_half:]
            d = mk(comm_l, send_l, recv_l, 0, k, left)
            d.start()
            descs[("l", 0, k)] = d

        for h in range(N_DEV - 1):
            c_r = (my - 2 - h) % N_DEV
            c_l = (my + 2 + h) % N_DEV
            last = h == N_DEV - 2
            for k in range(S):
                sub = slice(k * m_sub, (k + 1) * m_sub)
                loc_r = x_ref[pl.ds(c_r * m_per + k * m_sub, m_sub), :d_half]
                loc_l = x_ref[pl.ds(c_l * m_per + k * m_sub, m_sub), d_half:]
                if not last:
                    descs[("r", h, k)].wait_recv()
                    comm_r[h + 1, sub, :] = comm_r[h + 1, sub, :] + loc_r
                    d = mk(comm_r, send_r, recv_r, h + 1, k, right)
                    d.start()
                    descs[("r", h + 1, k)] = d

                    descs[("l", h, k)].wait_recv()
                    comm_l[h + 1, sub, :] = comm_l[h + 1, sub, :] + loc_l
                    d = mk(comm_l, send_l, recv_l, h + 1, k, left)
                    d.start()
                    descs[("l", h + 1, k)] = d
                else:
                    descs[("r", h, k)].wait_recv()
                    descs[("l", h, k)].wait_recv()
                    y = jnp.concatenate(
                        [comm_r[h + 1, sub, :] + loc_r,
                         comm_l[h + 1, sub, :] + loc_l],
                        axis=1,
                    )
                    inv = lax.rsqrt(
                        jnp.mean(y * y, axis=-1, keepdims=True) + 1e-6
                    )
                    out_ref[sub, :] = y * inv * g_ref[:, :]

        for d in descs.values():
            d.wait_send()

    return pl.pallas_call(
        body,
        out_shape=jax.ShapeDtypeStruct((m_per, D), jnp.float32),
        in_specs=[
            pl.BlockSpec(memory_space=pltpu.VMEM),
            pl.BlockSpec(memory_space=pltpu.VMEM),
        ],
        out_specs=pl.BlockSpec(memory_space=pltpu.VMEM),
        scratch_shapes=[
            pltpu.VMEM((N_DEV, m_per, d_half), jnp.float32),
            pltpu.VMEM((N_DEV, m_per, d_half), jnp.float32),
            pltpu.SemaphoreType.DMA((N_DEV - 1, S)),
            pltpu.SemaphoreType.DMA((N_DEV - 1, S)),
            pltpu.SemaphoreType.DMA((N_DEV - 1, S)),
            pltpu.SemaphoreType.DMA((N_DEV - 1, S)),
        ],
        compiler_params=pltpu.CompilerParams(collective_id=0),
    )(x, g)


# device time: 81381 ns/iter; 1.8851x vs baseline; 1.0005x over previous
import jax
import jax.numpy as jnp
from jax import lax
from jax.experimental import pallas as pl
from jax.experimental.pallas import tpu as pltpu

N_DEV = 4
S = 4


def kernel(partial, gamma):
    _, M, D = partial.shape
    x = partial.reshape(M, D)
    g = gamma.reshape(1, D)
    m_per = M // N_DEV
    d_half = D // 2
    m_sub = m_per // S

    def body(
        x_ref, g_ref, out_ref,
        comm_r, comm_l, send_r, recv_r, send_l, recv_l,
    ):
        my = lax.axis_index("i")
        left = (my - 1) % N_DEV
        right = (my + 1) % N_DEV

        barrier_sem = pltpu.get_barrier_semaphore()
        for nbr in (left, right):
            pl.semaphore_signal(
                barrier_sem, inc=1,
                device_id=(nbr,), device_id_type=pl.DeviceIdType.MESH,
            )
        pl.semaphore_wait(barrier_sem, 2)

        def mk(comm, sends, recvs, h, k, tgt, src=None):
            rows = pl.ds(k * m_sub, m_sub)
            return pltpu.make_async_remote_copy(
                src_ref=comm.at[h, rows] if src is None else src,
                dst_ref=comm.at[h + 1, rows],
                send_sem=sends.at[h, k],
                recv_sem=recvs.at[h, k],
                device_id=(tgt,),
                device_id_type=pl.DeviceIdType.MESH,
            )

        cr = (my - 1) % N_DEV
        cl = (my + 1) % N_DEV
        descs = {}
        for k in range(S):
            src = x_ref.at[pl.ds(cr * m_per + k * m_sub, m_sub), pl.ds(0, d_half)]
            d = mk(comm_r, send_r, recv_r, 0, k, right, src=src)
            d.start()
            descs[("r", 0, k)] = d
            src = x_ref.at[pl.ds(cl * m_per + k * m_sub, m_sub), pl.ds(d_half, d_half)]
            d = mk(comm_l, send_l, recv_l, 0, k, left, src=src)
            d.start()
            descs[("l", 0, k)] = d

        for h in range(N_DEV - 1):
            c_r = (my - 2 - h) % N_DEV
            c_l = (my + 2 + h) % N_DEV
            last = h == N_DEV - 2
            for k in range(S):
                sub = slice(k * m_sub, (k + 1) * m_sub)
                loc_r = x_ref[pl.ds(c_r * m_per + k * m_sub, m_sub), :d_half]
                loc_l = x_ref[pl.ds(c_l * m_per + k * m_sub, m_sub), d_half:]
                if not last:
                    descs[("r", h, k)].wait_recv()
                    comm_r[h + 1, sub, :] = comm_r[h + 1, sub, :] + loc_r
                    d = mk(comm_r, send_r, recv_r, h + 1, k, right)
                    d.start()
                    descs[("r", h + 1, k)] = d

                    descs[("l", h, k)].wait_recv()
                    comm_l[h + 1, sub, :] = comm_l[h + 1, sub, :] + loc_l
                    d = mk(comm_l, send_l, recv_l, h + 1, k, left)
                    d.start()
                    descs[("l", h + 1, k)] = d
                else:
                    descs[("r", h, k)].wait_recv()
                    descs[("l", h, k)].wait_recv()
                    y = jnp.concatenate(
                        [comm_r[h + 1, sub, :] + loc_r,
                         comm_l[h + 1, sub, :] + loc_l],
                        axis=1,
                    )
                    inv = lax.rsqrt(
                        jnp.mean(y * y, axis=-1, keepdims=True) + 1e-6
                    )
                    out_ref[sub, :] = y * inv * g_ref[:, :]

        for d in descs.values():
            d.wait_send()

    return pl.pallas_call(
        body,
        out_shape=jax.ShapeDtypeStruct((m_per, D), jnp.float32),
        in_specs=[
            pl.BlockSpec(memory_space=pltpu.VMEM),
            pl.BlockSpec(memory_space=pltpu.VMEM),
        ],
        out_specs=pl.BlockSpec(memory_space=pltpu.VMEM),
        scratch_shapes=[
            pltpu.VMEM((N_DEV, m_per, d_half), jnp.float32),
            pltpu.VMEM((N_DEV, m_per, d_half), jnp.float32),
            pltpu.SemaphoreType.DMA((N_DEV - 1, S)),
            pltpu.SemaphoreType.DMA((N_DEV - 1, S)),
            pltpu.SemaphoreType.DMA((N_DEV - 1, S)),
            pltpu.SemaphoreType.DMA((N_DEV - 1, S)),
        ],
        compiler_params=pltpu.CompilerParams(collective_id=0),
    )(x, g)
